# baseline (speedup 1.0000x reference)
import functools

import jax
import jax.numpy as jnp
from jax import lax
from jax.experimental import pallas as pl
from jax.experimental.pallas import tpu as pltpu
from jax.experimental.pallas import tpu_sc as plsc

BATCH = 1024
SEQ = 200
SEQP = 208
DIM = 128
NLANE = 16
NCHUNK = DIM // NLANE
HALF = SEQP // 2
NGRP = SEQP // NLANE

_info = plsc.get_sparse_core_info()
NC, NS = _info.num_cores, _info.num_subcores
NW = NC * NS
ROWS_PER_W = BATCH // NW

_mesh = plsc.VectorSubcoreMesh(core_axis_name="c", subcore_axis_name="s")


@functools.partial(
    pl.kernel,
    mesh=_mesh,
    out_type=jax.ShapeDtypeStruct((BATCH, DIM), jnp.float32),
    scratch_types=[
        pltpu.VMEM((ROWS_PER_W, SEQP), jnp.float32),
        pltpu.VMEM((ROWS_PER_W, 2, HALF), jnp.int32),
        pltpu.VMEM((2, SEQP, DIM), jnp.float32),
        pltpu.VMEM((ROWS_PER_W, DIM), jnp.float32),
        pltpu.SemaphoreType.DMA,
        pltpu.SemaphoreType.DMA,
    ],
)
def _bag_kernel(x_hbm, ind_hbm, table_hbm, out_hbm,
                w_v, idx_v, rows_v, out_v, sem0, sem1):
    wid = lax.axis_index("s") * NC + lax.axis_index("c")
    base = wid * ROWS_PER_W
    sems = (sem0, sem1)

    pltpu.sync_copy(ind_hbm.at[pl.ds(base, ROWS_PER_W)], idx_v)

    def issue(r, b):
        for h in range(2):
            pltpu.async_copy(
                table_hbm.at[idx_v.at[r, h]],
                rows_v.at[b, pl.ds(h * HALF, HALF)], sems[b],
            )

    def drain(b):
        pltpu.make_async_copy(
            table_hbm.at[pl.ds(0, SEQP)], rows_v.at[b], sems[b],
        ).wait()

    def compute(r, b):
        def grp_body(g, accs):
            w16 = w_v[r, pl.ds(g * NLANE, NLANE)]
            for j in range(NLANE):
                wj = w16[j]
                accs = tuple(
                    accs[c]
                    + wj * rows_v[b, g * NLANE + j, pl.ds(c * NLANE, NLANE)]
                    for c in range(NCHUNK)
                )
            return accs

        accs = lax.fori_loop(
            0, NGRP, grp_body,
            tuple(jnp.zeros((NLANE,), jnp.float32) for _ in range(NCHUNK)),
            unroll=4,
        )
        for c in range(NCHUNK):
            out_v[r, pl.ds(c * NLANE, NLANE)] = jnp.maximum(accs[c], 0.0)

    issue(0, 0)
    pltpu.sync_copy(x_hbm.at[pl.ds(base, ROWS_PER_W)], w_v)

    def outer(r0):
        for b in range(2):
            r = r0 + b

            @pl.when(r + 1 < ROWS_PER_W)
            def _():
                issue(r + 1, 1 - b)

            drain(b)
            compute(r, b)

    pl.loop(0, ROWS_PER_W, step=2)(outer)
    pltpu.sync_copy(out_v, out_hbm.at[pl.ds(base, ROWS_PER_W)])


def kernel(X, X_ind, emb_table):
    pad = SEQP - SEQ
    w = jnp.pad(X, ((0, 0), (0, pad)))
    ind_p = jnp.concatenate([X_ind, X_ind[:, :pad]], axis=1)
    ind3 = ind_p.reshape(BATCH, 2, HALF)
    return _bag_kernel(w, ind3, emb_table)


# speedup vs baseline: 1.4947x; 1.4947x over previous
import functools

import jax
import jax.numpy as jnp
from jax import lax
from jax.experimental import pallas as pl
from jax.experimental.pallas import tpu as pltpu
from jax.experimental.pallas import tpu_sc as plsc

BATCH = 1024
SEQ = 200
SEQP = 208
DIM = 128
NLANE = 16
NCHUNK = DIM // NLANE
HALF = SEQP // 2
NGRP = SEQ // NLANE
TAIL = SEQ - NGRP * NLANE

_info = plsc.get_sparse_core_info()
NC, NS = _info.num_cores, _info.num_subcores
NW = NC * NS
ROWS_PER_W = BATCH // NW

_mesh = plsc.VectorSubcoreMesh(core_axis_name="c", subcore_axis_name="s")


@functools.partial(
    pl.kernel,
    mesh=_mesh,
    out_type=jax.ShapeDtypeStruct((BATCH, DIM), jnp.float32),
    scratch_types=[
        pltpu.VMEM((ROWS_PER_W, SEQP), jnp.float32),
        pltpu.VMEM((ROWS_PER_W, 2, HALF), jnp.int32),
        pltpu.VMEM((4, SEQP, DIM), jnp.float32),
        pltpu.VMEM((ROWS_PER_W, DIM), jnp.float32),
        pltpu.SemaphoreType.DMA,
        pltpu.SemaphoreType.DMA,
        pltpu.SemaphoreType.DMA,
        pltpu.SemaphoreType.DMA,
    ],
)
def _bag_kernel(x_hbm, ind_hbm, table_hbm, out_hbm,
                w_v, idx_v, rows_v, out_v, sem0, sem1, sem2, sem3):
    wid = lax.axis_index("s") * NC + lax.axis_index("c")
    base = wid * ROWS_PER_W
    sems = (sem0, sem1, sem2, sem3)

    pltpu.sync_copy(ind_hbm.at[pl.ds(base, ROWS_PER_W)], idx_v)

    def issue(r, b):
        for h in range(2):
            pltpu.async_copy(
                table_hbm.at[idx_v.at[r, h]],
                rows_v.at[b, pl.ds(h * HALF, HALF)], sems[b],
            )

    def drain(b):
        pltpu.make_async_copy(
            table_hbm.at[pl.ds(0, SEQP)], rows_v.at[b], sems[b],
        ).wait()

    def compute(r, b):
        def grp_body(g, accs):
            w16 = w_v[r, pl.ds(g * NLANE, NLANE)]
            for j in range(NLANE):
                wj = w16[j]
                accs = tuple(
                    accs[c]
                    + wj * rows_v[b, g * NLANE + j, pl.ds(c * NLANE, NLANE)]
                    for c in range(NCHUNK)
                )
            return accs

        accs = lax.fori_loop(
            0, NGRP, grp_body,
            tuple(jnp.zeros((NLANE,), jnp.float32) for _ in range(NCHUNK)),
        )
        w16 = w_v[r, pl.ds(NGRP * NLANE, NLANE)]
        for j in range(TAIL):
            wj = w16[j]
            accs = tuple(
                accs[c]
                + wj * rows_v[b, NGRP * NLANE + j, pl.ds(c * NLANE, NLANE)]
                for c in range(NCHUNK)
            )
        for c in range(NCHUNK):
            out_v[r, pl.ds(c * NLANE, NLANE)] = jnp.maximum(accs[c], 0.0)

    issue(0, 0)
    issue(1, 1)
    issue(2, 2)
    pltpu.sync_copy(x_hbm.at[pl.ds(base, ROWS_PER_W)], w_v)

    def outer(r0):
        for b in range(4):
            r = r0 + b

            @pl.when(r + 3 < ROWS_PER_W)
            def _():
                issue(r + 3, (b + 3) % 4)

            drain(b)
            compute(r, b)

    pl.loop(0, ROWS_PER_W, step=4)(outer)
    pltpu.sync_copy(out_v, out_hbm.at[pl.ds(base, ROWS_PER_W)])


def kernel(X, X_ind, emb_table):
    pad = SEQP - SEQ
    w = jnp.pad(X, ((0, 0), (0, pad)))
    ind_p = jnp.concatenate([X_ind, X_ind[:, :pad]], axis=1)
    ind3 = ind_p.reshape(BATCH, 2, HALF)
    return _bag_kernel(w, ind3, emb_table)


# speedup vs baseline: 1.5549x; 1.0403x over previous
import functools

import jax
import jax.numpy as jnp
from jax import lax
from jax.experimental import pallas as pl
from jax.experimental.pallas import tpu as pltpu
from jax.experimental.pallas import tpu_sc as plsc

BATCH = 1024
SEQ = 200
SEQP = 208
DIM = 128
NLANE = 16
NCHUNK = DIM // NLANE
HALF = SEQP // 2
NGRP = SEQP // NLANE

_info = plsc.get_sparse_core_info()
NC, NS = _info.num_cores, _info.num_subcores
NW = NC * NS
ROWS_PER_W = BATCH // NW

_mesh = plsc.VectorSubcoreMesh(core_axis_name="c", subcore_axis_name="s")


@functools.partial(
    pl.kernel,
    mesh=_mesh,
    out_type=jax.ShapeDtypeStruct((BATCH, DIM), jnp.float32),
    scratch_types=[
        pltpu.VMEM((ROWS_PER_W, SEQP), jnp.float32),
        pltpu.VMEM((ROWS_PER_W, 2, HALF), jnp.int32),
        pltpu.VMEM((4, SEQP, DIM), jnp.float32),
        pltpu.VMEM((ROWS_PER_W, DIM), jnp.float32),
        pltpu.SemaphoreType.DMA,
        pltpu.SemaphoreType.DMA,
        pltpu.SemaphoreType.DMA,
        pltpu.SemaphoreType.DMA,
    ],
)
def _bag_kernel(x_hbm, ind_hbm, table_hbm, out_hbm,
                w_v, idx_v, rows_v, out_v, sem0, sem1, sem2, sem3):
    wid = lax.axis_index("s") * NC + lax.axis_index("c")
    base = wid * ROWS_PER_W
    sems = (sem0, sem1, sem2, sem3)

    pltpu.sync_copy(ind_hbm.at[pl.ds(base, ROWS_PER_W)], idx_v)

    def issue(r, b):
        for h in range(2):
            pltpu.async_copy(
                table_hbm.at[idx_v.at[r, h]],
                rows_v.at[b, pl.ds(h * HALF, HALF)], sems[b],
            )

    def drain(b):
        pltpu.make_async_copy(
            table_hbm.at[pl.ds(0, SEQP)], rows_v.at[b], sems[b],
        ).wait()

    def compute(r, b):
        def grp_body(g, accs):
            w16 = w_v[r, pl.ds(g * NLANE, NLANE)]
            for j in range(NLANE):
                wj = w16[j]
                accs = tuple(
                    accs[c]
                    + wj * rows_v[b, g * NLANE + j, pl.ds(c * NLANE, NLANE)]
                    for c in range(NCHUNK)
                )
            return accs

        accs = lax.fori_loop(
            0, NGRP, grp_body,
            tuple(jnp.zeros((NLANE,), jnp.float32) for _ in range(NCHUNK)),
        )
        for c in range(NCHUNK):
            out_v[r, pl.ds(c * NLANE, NLANE)] = jnp.maximum(accs[c], 0.0)

    issue(0, 0)
    issue(1, 1)
    issue(2, 2)
    pltpu.sync_copy(x_hbm.at[pl.ds(base, ROWS_PER_W)], w_v)

    def outer(r0):
        for b in range(4):
            r = r0 + b

            @pl.when(r + 3 < ROWS_PER_W)
            def _():
                issue(r + 3, (b + 3) % 4)

            drain(b)
            compute(r, b)

    pl.loop(0, ROWS_PER_W, step=4)(outer)
    pltpu.sync_copy(out_v, out_hbm.at[pl.ds(base, ROWS_PER_W)])


def kernel(X, X_ind, emb_table):
    pad = SEQP - SEQ
    w = jnp.pad(X, ((0, 0), (0, pad)))
    ind_p = jnp.concatenate([X_ind, X_ind[:, :pad]], axis=1)
    ind3 = ind_p.reshape(BATCH, 2, HALF)
    return _bag_kernel(w, ind3, emb_table)
